# Initial kernel scaffold; baseline (speedup 1.0000x reference)
import jax
import jax.numpy as jnp
from jax.experimental import pallas as pl


def kernel(x, W, iInd, jInd):
    raise NotImplementedError("write your pallas kernel here")


# speedup vs baseline: 6.8955x; 6.8955x over previous
import functools

import jax
import jax.numpy as jnp
from jax import lax
from jax.experimental import pallas as pl
from jax.experimental.pallas import tpu as pltpu, tpu_sc as plsc

C = 128
N = 10000
E = 320000

NUM_CORES = 2
NUM_SUBCORES = 16
NUM_WORKERS = NUM_CORES * NUM_SUBCORES
CPT = C // NUM_WORKERS

CHUNK = 4000
NUM_CHUNKS = E // CHUNK
STEPS = CHUNK // 16

_mesh = plsc.VectorSubcoreMesh(core_axis_name="c", subcore_axis_name="s")


@functools.partial(
    pl.kernel,
    out_type=jax.ShapeDtypeStruct((C * N,), jnp.float32),
    mesh=_mesh,
    scratch_types=[
        pltpu.VMEM((CPT * N,), jnp.float32),
        pltpu.VMEM((CPT * N,), jnp.float32),
        pltpu.VMEM((CHUNK,), jnp.int32),
        pltpu.VMEM((CHUNK,), jnp.int32),
        pltpu.VMEM((CHUNK,), jnp.float32),
    ],
    compiler_params=pltpu.CompilerParams(needs_layout_passes=False),
)
def _graph_lap(x_hbm, w_hbm, i_hbm, j_hbm, out_hbm, xl, acc, ib, jb, wb):
    cid = lax.axis_index("c")
    sid = lax.axis_index("s")
    wid = sid * NUM_CORES + cid
    base = wid * (CPT * N)

    pltpu.sync_copy(x_hbm.at[pl.ds(base, CPT * N)], xl)

    zeros = jnp.zeros((16,), jnp.float32)

    def _zero(k, _):
        acc[pl.ds(k * 16, 16)] = zeros
        return _

    lax.fori_loop(0, CPT * N // 16, _zero, None)

    def _chunk(cidx, _):
        off = cidx * CHUNK
        pltpu.sync_copy(i_hbm.at[pl.ds(off, CHUNK)], ib)
        pltpu.sync_copy(j_hbm.at[pl.ds(off, CHUNK)], jb)
        pltpu.sync_copy(w_hbm.at[pl.ds(off, CHUNK)], wb)

        def _step(s, _):
            e = s * 16
            iv = ib[pl.ds(e, 16)]
            jv = jb[pl.ds(e, 16)]
            wv = wb[pl.ds(e, 16)]
            w2 = wv * wv
            for ch in range(CPT):
                ai = iv + (ch * N)
                aj = jv + (ch * N)
                xi = plsc.load_gather(xl, [ai])
                xj = plsc.load_gather(xl, [aj])
                wf = w2 * (xj - xi)
                plsc.addupdate_scatter(acc, [ai], wf)
                plsc.addupdate_scatter(acc, [aj], -wf)
            return _

        lax.fori_loop(0, STEPS, _step, None)
        return _

    lax.fori_loop(0, NUM_CHUNKS, _chunk, None)

    pltpu.sync_copy(acc, out_hbm.at[pl.ds(base, CPT * N)])


def kernel(x, W, iInd, jInd):
    xf = x.reshape(C * N)
    out = _graph_lap(xf, W.reshape(E), iInd.reshape(E), jInd.reshape(E))
    return out.reshape(1, C, N)


# speedup vs baseline: 13.8276x; 2.0053x over previous
import functools

import jax
import jax.numpy as jnp
from jax import lax
from jax.experimental import pallas as pl
from jax.experimental.pallas import tpu as pltpu, tpu_sc as plsc

C = 128
N = 10000
E = 320000

NUM_CORES = 2
NUM_SUBCORES = 16
NUM_WORKERS = NUM_CORES * NUM_SUBCORES
CPT = C // NUM_WORKERS

CHUNK = 6400
NUM_CHUNKS = E // CHUNK
STEPS = CHUNK // 16
PAIRS = NUM_CHUNKS // 2

_mesh = plsc.VectorSubcoreMesh(core_axis_name="c", subcore_axis_name="s")


@functools.partial(
    pl.kernel,
    out_type=jax.ShapeDtypeStruct((C * N,), jnp.float32),
    mesh=_mesh,
    scratch_types=[
        pltpu.VMEM((CPT * N,), jnp.float32),
        pltpu.VMEM((CPT * N,), jnp.float32),
        pltpu.VMEM((CHUNK,), jnp.int32),
        pltpu.VMEM((CHUNK,), jnp.int32),
        pltpu.VMEM((CHUNK,), jnp.float32),
        pltpu.VMEM((CHUNK,), jnp.int32),
        pltpu.VMEM((CHUNK,), jnp.int32),
        pltpu.VMEM((CHUNK,), jnp.float32),
        pltpu.SemaphoreType.DMA,
        pltpu.SemaphoreType.DMA,
    ],
    compiler_params=pltpu.CompilerParams(needs_layout_passes=False),
)
def _graph_lap(x_hbm, w_hbm, i_hbm, j_hbm, out_hbm,
               xl, acc, ib0, jb0, wb0, ib1, jb1, wb1, sem0, sem1):
    cid = lax.axis_index("c")
    sid = lax.axis_index("s")
    wid = sid * NUM_CORES + cid
    base = wid * (CPT * N)

    pltpu.sync_copy(x_hbm.at[pl.ds(base, CPT * N)], xl)

    pltpu.async_copy(i_hbm.at[pl.ds(0, CHUNK)], ib0, sem0)
    pltpu.async_copy(j_hbm.at[pl.ds(0, CHUNK)], jb0, sem0)
    pltpu.async_copy(w_hbm.at[pl.ds(0, CHUNK)], wb0, sem0)

    zeros = jnp.zeros((16,), jnp.float32)

    def _zero(k, carry):
        acc[pl.ds(k * 16, 16)] = zeros
        return carry

    lax.fori_loop(0, CPT * N // 16, _zero, None)

    def _issue(off, ibuf, jbuf, wbuf, sem):
        pltpu.async_copy(i_hbm.at[pl.ds(off, CHUNK)], ibuf, sem)
        pltpu.async_copy(j_hbm.at[pl.ds(off, CHUNK)], jbuf, sem)
        pltpu.async_copy(w_hbm.at[pl.ds(off, CHUNK)], wbuf, sem)

    def _drain(ibuf, jbuf, wbuf, sem):
        pltpu.make_async_copy(i_hbm.at[pl.ds(0, CHUNK)], ibuf, sem).wait()
        pltpu.make_async_copy(j_hbm.at[pl.ds(0, CHUNK)], jbuf, sem).wait()
        pltpu.make_async_copy(w_hbm.at[pl.ds(0, CHUNK)], wbuf, sem).wait()

    def _process(ibuf, jbuf, wbuf):
        def _step(s, carry):
            e = s * 16
            iv = ibuf[pl.ds(e, 16)]
            jv = jbuf[pl.ds(e, 16)]
            wv = wbuf[pl.ds(e, 16)]
            w2 = wv * wv
            ais = [iv + (ch * N) for ch in range(1, CPT)]
            ajs = [jv + (ch * N) for ch in range(1, CPT)]
            ais.insert(0, iv)
            ajs.insert(0, jv)
            xis = [plsc.load_gather(xl, [a]) for a in ais]
            xjs = [plsc.load_gather(xl, [a]) for a in ajs]
            wfs = [w2 * (xjs[ch] - xis[ch]) for ch in range(CPT)]
            negs = [-wfs[ch] for ch in range(CPT)]
            for ch in range(CPT):
                plsc.addupdate_scatter(acc, [ais[ch]], wfs[ch])
                plsc.addupdate_scatter(acc, [ajs[ch]], negs[ch])
            return carry

        lax.fori_loop(0, STEPS, _step, None)

    def _pair(p, carry):
        _drain(ib0, jb0, wb0, sem0)
        _issue((2 * p + 1) * CHUNK, ib1, jb1, wb1, sem1)
        _process(ib0, jb0, wb0)
        _drain(ib1, jb1, wb1, sem1)

        @pl.when(p + 1 < PAIRS)
        def _():
            _issue((2 * p + 2) * CHUNK, ib0, jb0, wb0, sem0)

        _process(ib1, jb1, wb1)
        return carry

    lax.fori_loop(0, PAIRS, _pair, None)

    pltpu.sync_copy(acc, out_hbm.at[pl.ds(base, CPT * N)])


def kernel(x, W, iInd, jInd):
    xf = x.reshape(C * N)
    out = _graph_lap(xf, W.reshape(E), iInd.reshape(E), jInd.reshape(E))
    return out.reshape(1, C, N)
